# Initial kernel scaffold; baseline (speedup 1.0000x reference)
import jax
import jax.numpy as jnp
from jax.experimental import pallas as pl


def kernel(query_reps, support_reps, target_ids):
    raise NotImplementedError("write your pallas kernel here")


# speedup vs baseline: 1.7303x; 1.7303x over previous
import jax
import jax.numpy as jnp
from jax.experimental import pallas as pl

_Q = 16384
_K = 256
_D = 128
_TQ = 1024


def _head_kernel(q_ref, s_ref, t_ref, correct_ref, loss_ref):
    i = pl.program_id(0)
    g = pl.num_programs(0)
    q = q_ref[...]
    s = s_ref[...]
    scores = jax.lax.dot_general(
        q, s, (((1,), (1,)), ((), ())), preferred_element_type=jnp.float32
    )
    t = t_ref[0, :, :]
    iota = jax.lax.broadcasted_iota(jnp.int32, scores.shape, 1)
    m = jnp.max(scores, axis=1, keepdims=True)
    preds = jnp.min(jnp.where(scores == m, iota, _K), axis=1, keepdims=True)
    correct_ref[0, :, :] = (preds == t).astype(jnp.int8)
    lse = m + jnp.log(jnp.sum(jnp.exp(scores - m), axis=1, keepdims=True))
    tgt = jnp.sum(jnp.where(iota == t, scores, 0.0), axis=1, keepdims=True)
    nll_sum = jnp.sum(lse - tgt)
    prev = jnp.where(i == 0, jnp.zeros_like(loss_ref[...]), loss_ref[...])
    acc = prev + nll_sum
    loss_ref[...] = jnp.where(i == g - 1, acc / _Q, acc)


@jax.jit
def kernel(query_reps, support_reps, target_ids):
    grid = _Q // _TQ
    targets = target_ids.astype(jnp.int32).reshape(grid, _TQ, 1)
    correct8, loss = pl.pallas_call(
        _head_kernel,
        grid=(grid,),
        in_specs=[
            pl.BlockSpec((_TQ, _D), lambda i: (i, 0)),
            pl.BlockSpec((_K, _D), lambda i: (0, 0)),
            pl.BlockSpec((1, _TQ, 1), lambda i: (i, 0, 0)),
        ],
        out_specs=[
            pl.BlockSpec((1, _TQ, 1), lambda i: (i, 0, 0)),
            pl.BlockSpec((1, 128), lambda i: (0, 0)),
        ],
        out_shape=[
            jax.ShapeDtypeStruct((grid, _TQ, 1), jnp.int8),
            jax.ShapeDtypeStruct((1, 128), jnp.float32),
        ],
    )(query_reps, support_reps, targets)
    return (loss[0, 0], correct8.reshape(_Q).astype(jnp.bool_))


# speedup vs baseline: 3.6313x; 2.0987x over previous
import jax
import jax.numpy as jnp
from jax.experimental import pallas as pl

_Q = 16384
_K = 256
_D = 128
_TQ = 1024


def _head_kernel(q_ref, s_ref, t_ref, correct_ref, loss_ref):
    i = pl.program_id(0)
    g = pl.num_programs(0)
    q = q_ref[...]
    s = s_ref[...]
    scores = jax.lax.dot_general(
        s, q, (((1,), (1,)), ((), ())), preferred_element_type=jnp.float32
    )
    t = t_ref[0, :, :]
    iota = jax.lax.broadcasted_iota(jnp.int32, scores.shape, 0)
    m = jnp.max(scores, axis=0, keepdims=True)
    eqm = scores == m
    tgt = jnp.sum(jnp.where(iota == t, scores, 0.0), axis=0, keepdims=True)
    bad = jnp.sum(jnp.where(eqm & (iota < t), 1.0, 0.0), axis=0, keepdims=True)
    correct_ref[0, :, :] = ((tgt == m) & (bad == 0.0)).astype(jnp.int8)
    lse = m + jnp.log(jnp.sum(jnp.exp(scores - m), axis=0, keepdims=True))
    nll_sum = jnp.sum(lse - tgt)
    prev = jnp.where(i == 0, jnp.zeros_like(loss_ref[...]), loss_ref[...])
    acc = prev + nll_sum
    loss_ref[...] = jnp.where(i == g - 1, acc / _Q, acc)


@jax.jit
def kernel(query_reps, support_reps, target_ids):
    grid = _Q // _TQ
    targets = target_ids.astype(jnp.int32).reshape(grid, 1, _TQ)
    correct8, loss = pl.pallas_call(
        _head_kernel,
        grid=(grid,),
        in_specs=[
            pl.BlockSpec((_TQ, _D), lambda i: (i, 0)),
            pl.BlockSpec((_K, _D), lambda i: (0, 0)),
            pl.BlockSpec((1, 1, _TQ), lambda i: (i, 0, 0)),
        ],
        out_specs=[
            pl.BlockSpec((1, 1, _TQ), lambda i: (i, 0, 0)),
            pl.BlockSpec((1, 128), lambda i: (0, 0)),
        ],
        out_shape=[
            jax.ShapeDtypeStruct((grid, 1, _TQ), jnp.int8),
            jax.ShapeDtypeStruct((1, 128), jnp.float32),
        ],
    )(query_reps, support_reps, targets)
    return (loss[0, 0], correct8.reshape(_Q).astype(jnp.bool_))


# speedup vs baseline: 4.5210x; 1.2450x over previous
import jax
import jax.numpy as jnp
from jax.experimental import pallas as pl

_Q = 16384
_K = 256
_D = 128
_TQ = 4096


def _head_kernel(q_ref, s_ref, t_ref, correct_ref, loss_ref):
    i = pl.program_id(0)
    g = pl.num_programs(0)
    q = q_ref[...]
    s = s_ref[...]
    scores = jax.lax.dot_general(
        s, q, (((1,), (1,)), ((), ())), preferred_element_type=jnp.float32
    )
    t = t_ref[0, :, :]
    iota = jax.lax.broadcasted_iota(jnp.int32, scores.shape, 0)
    m = jnp.max(scores, axis=0, keepdims=True)
    tgt = jnp.sum(jnp.where(iota == t, scores, 0.0), axis=0, keepdims=True)
    preds = jnp.min(jnp.where(scores == m, iota, _K), axis=0, keepdims=True)
    correct_ref[0, :, :] = (preds == t).astype(jnp.int8)
    lse = m + jnp.log(jnp.sum(jnp.exp(scores - m), axis=0, keepdims=True))
    nll_sum = jnp.sum(lse - tgt)
    prev = jnp.where(i == 0, jnp.zeros_like(loss_ref[...]), loss_ref[...])
    acc = prev + nll_sum
    loss_ref[...] = jnp.where(i == g - 1, acc / _Q, acc)


@jax.jit
def kernel(query_reps, support_reps, target_ids):
    grid = _Q // _TQ
    targets = target_ids.astype(jnp.int32).reshape(grid, 1, _TQ)
    correct8, loss = pl.pallas_call(
        _head_kernel,
        grid=(grid,),
        in_specs=[
            pl.BlockSpec((_TQ, _D), lambda i: (i, 0)),
            pl.BlockSpec((_K, _D), lambda i: (0, 0)),
            pl.BlockSpec((1, 1, _TQ), lambda i: (i, 0, 0)),
        ],
        out_specs=[
            pl.BlockSpec((1, 1, _TQ), lambda i: (i, 0, 0)),
            pl.BlockSpec((1, 128), lambda i: (0, 0)),
        ],
        out_shape=[
            jax.ShapeDtypeStruct((grid, 1, _TQ), jnp.int8),
            jax.ShapeDtypeStruct((1, 128), jnp.float32),
        ],
    )(query_reps, support_reps, targets)
    return (loss[0, 0], correct8.reshape(_Q).astype(jnp.bool_))
